# baseline (speedup 1.0000x reference)
import functools

import jax
import jax.numpy as jnp
from jax import lax
from jax.experimental import pallas as pl
from jax.experimental.pallas import tpu as pltpu
from jax.experimental.pallas import tpu_sc as plsc

NUM_TABLES = 26
VOCAB = 100000
DIM = 32
BATCH = 16384

NC = 2
NS = 16
L = 16
NW = NC * NS

ROWS = NUM_TABLES * VOCAB
TOTAL = NUM_TABLES * BATCH

CHUNK = 1024
CHUNKS_PER_TABLE = BATCH // CHUNK
N_CHUNKS = TOTAL // (NW * CHUNK)
GROUPS = CHUNK // L


def _sc_lookup(ids_flat, tab_flat):
    mesh = plsc.VectorSubcoreMesh(core_axis_name="c", subcore_axis_name="s")

    @functools.partial(
        pl.kernel,
        mesh=mesh,
        compiler_params=pltpu.CompilerParams(
            use_tc_tiling_on_sc=False, needs_layout_passes=False
        ),
        out_type=jax.ShapeDtypeStruct((TOTAL * DIM,), jnp.float32),
        scratch_types=[
            pltpu.VMEM((CHUNK,), jnp.int32),
            pltpu.VMEM((CHUNK * DIM,), jnp.int32),
            pltpu.VMEM((CHUNK * DIM,), jnp.float32),
            pltpu.SemaphoreType.DMA,
        ],
    )
    def k(ids_hbm, tab_hbm, out_hbm, flat_v, eidx_v, buf_v, sem):
        wid = lax.axis_index("s") * NC + lax.axis_index("c")
        iota = lax.iota(jnp.int32, L)

        def chunk_body(j, carry):
            c = wid * N_CHUNKS + j
            base = c * CHUNK
            t = c // CHUNKS_PER_TABLE
            off = t * VOCAB
            pltpu.sync_copy(ids_hbm.at[pl.ds(base, CHUNK)], flat_v)

            @plsc.parallel_loop(0, GROUPS, unroll=4)
            def add_body(v):
                flat_v[pl.ds(v * L, L)] = flat_v[pl.ds(v * L, L)] + off

            @plsc.parallel_loop(0, CHUNK, unroll=4)
            def exp_body(n):
                jb = jnp.zeros((L,), jnp.int32) + n
                bc = plsc.load_gather(flat_v, [jb]) * DIM + iota
                eidx_v[pl.ds(n * DIM, L)] = bc
                eidx_v[pl.ds(n * DIM + L, L)] = bc + L

            cp = pltpu.make_async_copy(tab_hbm.at[eidx_v], buf_v, sem)
            cp.start()
            cp.wait()
            pltpu.sync_copy(buf_v, out_hbm.at[pl.ds(base * DIM, CHUNK * DIM)])
            return carry

        lax.fori_loop(0, N_CHUNKS, chunk_body, 0)

    return k(ids_flat, tab_flat)


def kernel(ids, tables):
    out_flat = _sc_lookup(
        ids.reshape(TOTAL),
        tables.reshape(ROWS * DIM),
    )
    return out_flat.reshape(NUM_TABLES, BATCH, DIM)


# speedup vs baseline: 1.3876x; 1.3876x over previous
import functools

import jax
import jax.numpy as jnp
from jax import lax
from jax.experimental import pallas as pl
from jax.experimental.pallas import tpu as pltpu
from jax.experimental.pallas import tpu_sc as plsc

NUM_TABLES = 26
VOCAB = 100000
DIM = 32
BATCH = 16384

NC = 2
NS = 16
L = 16
NW = NC * NS

TOTAL = NUM_TABLES * BATCH

CHUNK = 1024
GATHER = 128
N_GATHER = CHUNK // GATHER
CHUNKS_PER_TABLE = BATCH // CHUNK
N_CHUNKS = TOTAL // (NW * CHUNK)


def _sc_lookup(ids, tables):
    mesh = plsc.VectorSubcoreMesh(core_axis_name="c", subcore_axis_name="s")

    @functools.partial(
        pl.kernel,
        mesh=mesh,
        compiler_params=pltpu.CompilerParams(
            use_tc_tiling_on_sc=False, needs_layout_passes=False
        ),
        out_type=jax.ShapeDtypeStruct((NUM_TABLES, BATCH, DIM), jnp.float32),
        scratch_types=[
            pltpu.VMEM((N_GATHER, GATHER), jnp.int32),
            pltpu.VMEM((CHUNK, DIM), jnp.float32),
            pltpu.SemaphoreType.DMA,
        ],
    )
    def k(ids_hbm, tab_hbm, out_hbm, idx_v, rows_v, sem):
        wid = lax.axis_index("s") * NC + lax.axis_index("c")

        def chunk_body(j, carry):
            c = wid * N_CHUNKS + j
            t = c // CHUNKS_PER_TABLE
            p = (c % CHUNKS_PER_TABLE) * CHUNK

            id_copies = []
            for q in range(N_GATHER):
                id_copies.append(
                    pltpu.make_async_copy(
                        ids_hbm.at[t, pl.ds(p + q * GATHER, GATHER)],
                        idx_v.at[q],
                        sem,
                    )
                )
                id_copies[-1].start()
            for cp in id_copies:
                cp.wait()

            copies = []
            for q in range(N_GATHER):
                copies.append(
                    pltpu.make_async_copy(
                        tab_hbm.at[t].at[idx_v.at[q]],
                        rows_v.at[pl.ds(q * GATHER, GATHER)],
                        sem,
                    )
                )
                copies[-1].start()
            for cp in copies:
                cp.wait()

            pltpu.sync_copy(rows_v, out_hbm.at[t, pl.ds(p, CHUNK)])
            return carry

        lax.fori_loop(0, N_CHUNKS, chunk_body, 0)

    return k(ids, tables)


def kernel(ids, tables):
    return _sc_lookup(ids, tables)
